# baseline (speedup 1.0000x reference)
import functools

import jax
import jax.numpy as jnp
from jax.experimental import pallas as pl


def _router_logits_body(x_ref, wr_ref, br_ref, out_ref):
    out_ref[...] = (
        jnp.dot(x_ref[...], wr_ref[...], preferred_element_type=jnp.float32)
        + br_ref[...]
    )


def _softmax_body(l_ref, p_ref):
    l = l_ref[...]
    m = jnp.max(l, axis=0, keepdims=True)
    e = jnp.exp(l - m)
    s = jnp.sum(e, axis=0, keepdims=True)
    p_ref[...] = e / s


def _ffn_body(xs_ref, w1_ref, b1_ref, w2_ref, b2_ref, s_ref, out_ref):
    xs = xs_ref[0].astype(jnp.bfloat16)
    w1 = w1_ref[0].astype(jnp.bfloat16)
    h = jnp.dot(xs, w1, preferred_element_type=jnp.float32) + b1_ref[0]
    h = jnp.maximum(h, 0.0).astype(jnp.bfloat16)
    w2 = w2_ref[0].astype(jnp.bfloat16)
    y = jnp.dot(h, w2, preferred_element_type=jnp.float32) + b2_ref[0]
    out_ref[0] = y * s_ref[0, 0][:, None]


def kernel(x, Wr, br, W1, b1, W2, b2):
    B, D = x.shape
    E = Wr.shape[1]
    H = W1.shape[2]
    O = W2.shape[2]
    C = min(512, B)

    RB = min(1024, B)
    logits = pl.pallas_call(
        _router_logits_body,
        grid=(B // RB,),
        in_specs=[
            pl.BlockSpec((RB, D), lambda i: (i, 0)),
            pl.BlockSpec((D, E), lambda i: (0, 0)),
            pl.BlockSpec((1, E), lambda i: (0, 0)),
        ],
        out_specs=pl.BlockSpec((RB, E), lambda i: (i, 0)),
        out_shape=jax.ShapeDtypeStruct((B, E), jnp.float32),
    )(x, Wr, br.reshape(1, E))

    probs = pl.pallas_call(
        _softmax_body,
        out_shape=jax.ShapeDtypeStruct((B, E), jnp.float32),
    )(logits)

    top_scores, top_idx = jax.lax.top_k(probs.T, C)

    xs = x[top_idx]

    y_w = pl.pallas_call(
        _ffn_body,
        grid=(E,),
        in_specs=[
            pl.BlockSpec((1, C, D), lambda e: (e, 0, 0)),
            pl.BlockSpec((1, D, H), lambda e: (e, 0, 0)),
            pl.BlockSpec((1, 1, H), lambda e: (e, 0, 0)),
            pl.BlockSpec((1, H, O), lambda e: (e, 0, 0)),
            pl.BlockSpec((1, 1, O), lambda e: (e, 0, 0)),
            pl.BlockSpec((1, 1, C), lambda e: (e, 0, 0)),
        ],
        out_specs=pl.BlockSpec((1, C, O), lambda e: (e, 0, 0)),
        out_shape=jax.ShapeDtypeStruct((E, C, O), jnp.float32),
    )(xs, W1, b1.reshape(E, 1, H), W2, b2.reshape(E, 1, O),
      top_scores.reshape(E, 1, C))

    flat_idx = top_idx.reshape(-1)
    out = jnp.zeros((B, O), dtype=x.dtype).at[flat_idx].add(y_w.reshape(-1, O))
    eid = jnp.repeat(jnp.arange(E), C)
    expert_mask = jnp.zeros((B, E), dtype=x.dtype).at[flat_idx, eid].set(1.0)
    return (out, probs, expert_mask)
